# baseline (speedup 1.0000x reference)
import functools

import jax
import jax.numpy as jnp
import numpy as np
from jax import lax
from jax.experimental import pallas as pl
from jax.experimental.pallas import tpu as pltpu
from jax.experimental.pallas import tpu_sc as plsc

N_UNIT = 4096
N_TREATED = 512
BATCH = 1024
HID = 64
T = 50
D = 16
DY = 8
TAU = 1.0
REG_B = 0.1
LAM_EXPRESS = 1.0
LAM_RECON = 0.5
LAM_PROG = 1.0

_BB = 128
_NSTEPS = BATCH // _BB

_NW = 32
_RPW = BATCH // _NW
_CHUNK = 8
_NCHUNK = _RPW // _CHUNK


def _rotl32(x, r):
    return ((x << np.uint32(r)) | (x >> np.uint32(32 - r))).astype(np.uint32)


def _gumbel_noise():
    n = BATCH * N_UNIT
    k1, k2 = np.uint32(0), np.uint32(42)
    ks = [k1, k2, np.uint32(k1 ^ k2 ^ np.uint32(0x1BD11BDA))]
    x = [np.zeros(n, dtype=np.uint32), np.arange(n, dtype=np.uint32)]
    rotations = [[13, 15, 26, 6], [17, 29, 16, 24]]
    x[0] += ks[0]
    x[1] += ks[1]
    for i in range(5):
        for r in rotations[i % 2]:
            x[0] += x[1]
            x[1] = _rotl32(x[1], r)
            x[1] ^= x[0]
        x[0] += ks[(i + 1) % 3]
        x[1] += ks[(i + 2) % 3] + np.uint32(i + 1)
    bits = x[0] ^ x[1]
    fl = ((bits >> np.uint32(9)) | np.uint32(0x3F800000)).view(np.float32) \
        - np.float32(1.0)
    mn, mx = np.float32(1e-6), np.float32(1.0 - 1e-6)
    u = np.maximum(mn, fl * (mx - mn) + mn).astype(np.float64)
    return (-np.log(-np.log(u))).astype(np.float32).reshape(BATCH, N_UNIT)


_G = _gumbel_noise()


def _sc_gather_body(table_hbm, idx_hbm, out_hbm, idx_v, rows_v, sems):
    wid = lax.axis_index("s") * 2 + lax.axis_index("c")
    base = pl.multiple_of(wid * _RPW, _RPW)
    pltpu.sync_copy(idx_hbm.at[pl.ds(base, _RPW)], idx_v)

    def _start(c):
        slot = c % 2
        pltpu.make_async_copy(table_hbm.at[idx_v.at[pl.ds(c * _CHUNK, _CHUNK)]],
                              rows_v.at[slot], sems.at[slot]).start()

    def _wait(c):
        slot = c % 2
        pltpu.make_async_copy(table_hbm.at[idx_v.at[pl.ds(c * _CHUNK, _CHUNK)]],
                              rows_v.at[slot], sems.at[slot]).wait()

    _start(0)
    for c in range(_NCHUNK):
        if c + 1 < _NCHUNK:
            _start(c + 1)
        _wait(c)
        pltpu.sync_copy(rows_v.at[c % 2],
                        out_hbm.at[pl.ds(base + c * _CHUNK, _CHUNK)])


def _sc_gather(B, batch_ind):
    k = pl.kernel(
        _sc_gather_body,
        out_type=jax.ShapeDtypeStruct((BATCH, N_UNIT), jnp.float32),
        mesh=plsc.VectorSubcoreMesh(core_axis_name="c", subcore_axis_name="s",
                                    num_cores=2, num_subcores=16),
        scratch_types=[
            pltpu.VMEM((_RPW,), jnp.int32),
            pltpu.VMEM((2, _CHUNK, N_UNIT), jnp.float32),
            pltpu.SemaphoreType.DMA((2,)),
        ],
    )
    return k(B, batch_ind)


_EB = 128
_ENSTEPS = BATCH // _EB


def _encoder_kernel(x_ref, t_ref, m_ref, wenc_ref, benc_ref,
                    c_ref, s1_ref, s2_ref, sm_ref, sq_ref):
    i = pl.program_id(0)

    xb = x_ref[...].reshape(T * _EB, D)
    tb = t_ref[...].reshape(T * _EB, 1)
    mb = m_ref[...].reshape(T * _EB, 1)
    m2 = mb * mb
    h = jnp.tanh(jnp.dot(xb, wenc_ref[...], preferred_element_type=jnp.float32)
                 + benc_ref[...] + 0.1 * tb) * mb
    sm = jnp.sum(mb.reshape(T, _EB, 1), axis=0)
    c_ref[...] = (jnp.sum(h.reshape(T, _EB, HID), axis=0)
                  / (sm + 1e-8))
    s1_ref[...] = jnp.sum((xb * m2).reshape(T, _EB, D), axis=0)
    sm_ref[...] = sm
    sq_ref[...] = jnp.sum(m2.reshape(T, _EB, 1), axis=0)

    @pl.when(i == 0)
    def _init():
        s2_ref[...] = jnp.zeros_like(s2_ref)
    s2_ref[...] += jnp.sum(xb * xb * m2, keepdims=True)


def _mix_kernel(bg_ref,
                g_ref,
                ind1_ref,
                c_ref,
                s1_ref,
                sm_ref,
                sq_ref,
                c0_ref,
                yc_ref,
                yb_ref,
                ym_ref,
                wdec_ref,
                bdec_ref,
                err_ref, reg_ref, prog_ref, rec_ref, ym_s_ref, sm_s_ref):
    i = pl.program_id(0)

    @pl.when(i == 0)
    def _init():
        for r in (err_ref, reg_ref, prog_ref, rec_ref, ym_s_ref, sm_s_ref):
            r[...] = jnp.zeros_like(r)

    x_dec = jnp.tanh(jnp.dot(c_ref[...], wdec_ref[...],
                             preferred_element_type=jnp.float32) + bdec_ref[...])
    rec_ref[...] += jnp.sum(-2.0 * s1_ref[...] * x_dec
                            + sq_ref[...] * x_dec * x_dec, keepdims=True)
    ym_s_ref[...] += jnp.sum(ym_ref[...], keepdims=True)
    sm_s_ref[...] += jnp.sum(sm_ref[...], keepdims=True)

    logits = (bg_ref[...] + g_ref[...]) * (1.0 / TAU)
    col = jax.lax.broadcasted_iota(jnp.int32, (_BB, N_UNIT), 1)
    logits = jnp.where(col == ind1_ref[...], jnp.float32(-1e30), logits)
    mx = jnp.max(logits, axis=1, keepdims=True)
    e = jnp.exp(logits - mx)
    s = jnp.sum(e, axis=1, keepdims=True)
    p = e * (1.0 / s)
    reg_ref[...] += jnp.sum(p * p, keepdims=True)
    pc0 = jnp.dot(p, c0_ref[...], preferred_element_type=jnp.float32)
    err = c_ref[...] - pc0
    err_ref[...] += jnp.sum(err * err, keepdims=True)
    yh = jnp.dot(p, yc_ref[...], preferred_element_type=jnp.float32)
    dy = yb_ref[...] - yh
    prog_ref[...] += jnp.sum(dy * dy * ym_ref[...], keepdims=True)


def kernel(x, t, mask, batch_ind, y_batch, y_control, y_mask, B, C0,
           W_enc, b_enc, W_dec, b_dec):
    f32 = jnp.float32
    B_gathered = _sc_gather(B, batch_ind)

    C, S1, S2, Sm, Sq = pl.pallas_call(
        _encoder_kernel,
        grid=(_ENSTEPS,),
        in_specs=[
            pl.BlockSpec((T, _EB, D), lambda i: (0, i, 0)),
            pl.BlockSpec((T, _EB, 1), lambda i: (0, i, 0)),
            pl.BlockSpec((T, _EB, 1), lambda i: (0, i, 0)),
            pl.BlockSpec((D, HID), lambda i: (0, 0)),
            pl.BlockSpec((1, HID), lambda i: (0, 0)),
        ],
        out_specs=[
            pl.BlockSpec((_EB, HID), lambda i: (i, 0)),
            pl.BlockSpec((_EB, D), lambda i: (i, 0)),
            pl.BlockSpec((1, 1), lambda i: (0, 0)),
            pl.BlockSpec((_EB, 1), lambda i: (i, 0)),
            pl.BlockSpec((_EB, 1), lambda i: (i, 0)),
        ],
        out_shape=[
            jax.ShapeDtypeStruct((BATCH, HID), f32),
            jax.ShapeDtypeStruct((BATCH, D), f32),
            jax.ShapeDtypeStruct((1, 1), f32),
            jax.ShapeDtypeStruct((BATCH, 1), f32),
            jax.ShapeDtypeStruct((BATCH, 1), f32),
        ],
    )(x, t, mask, W_enc, b_enc.reshape(1, HID))

    g = _G
    wrap = jnp.any(batch_ind >= N_UNIT)
    ind1 = jnp.where(wrap, batch_ind % (N_UNIT + 1), batch_ind)
    ind1m = jnp.where(ind1 < N_UNIT, ind1, -1).astype(jnp.int32).reshape(BATCH, 1)

    outs = pl.pallas_call(
        _mix_kernel,
        grid=(_NSTEPS,),
        in_specs=[
            pl.BlockSpec((_BB, N_UNIT), lambda i: (i, 0)),
            pl.BlockSpec((_BB, N_UNIT), lambda i: (i, 0)),
            pl.BlockSpec((_BB, 1), lambda i: (i, 0)),
            pl.BlockSpec((_BB, HID), lambda i: (i, 0)),
            pl.BlockSpec((_BB, D), lambda i: (i, 0)),
            pl.BlockSpec((_BB, 1), lambda i: (i, 0)),
            pl.BlockSpec((_BB, 1), lambda i: (i, 0)),
            pl.BlockSpec((N_UNIT, HID), lambda i: (0, 0)),
            pl.BlockSpec((N_UNIT, DY), lambda i: (0, 0)),
            pl.BlockSpec((_BB, DY), lambda i: (i, 0)),
            pl.BlockSpec((_BB, 1), lambda i: (i, 0)),
            pl.BlockSpec((HID, D), lambda i: (0, 0)),
            pl.BlockSpec((1, D), lambda i: (0, 0)),
        ],
        out_specs=[pl.BlockSpec((1, 1), lambda i: (0, 0))] * 6,
        out_shape=[jax.ShapeDtypeStruct((1, 1), f32)] * 6,
    )(B_gathered, g, ind1m, C, S1, Sm, Sq, C0, y_control, y_batch,
      y_mask.reshape(BATCH, 1), W_dec, b_dec.reshape(1, D))
    err_s, reg_s, prog_s, rec_s, ym_s, sm_s = [o[0, 0] for o in outs]

    err_mse = err_s / (BATCH * HID)
    reg = reg_s / (BATCH * N_UNIT)
    sel = LAM_EXPRESS * (err_mse + REG_B * reg)
    rec = (S2[0, 0] + rec_s) / sm_s * LAM_RECON
    prog = prog_s / ym_s * LAM_PROG
    return sel + rec + prog


# speedup vs baseline: 1.1304x; 1.1304x over previous
import jax
import jax.numpy as jnp
import numpy as np
from jax import lax
from jax.experimental import pallas as pl
from jax.experimental.pallas import tpu as pltpu
from jax.experimental.pallas import tpu_sc as plsc

N_UNIT = 4096
N_TREATED = 512
BATCH = 1024
HID = 64
T = 50
D = 16
DY = 8
TAU = 1.0
REG_B = 0.1
LAM_EXPRESS = 1.0
LAM_RECON = 0.5
LAM_PROG = 1.0

_BB = 128
_NSTEPS = BATCH // _BB

_NW = 32
_RPW = BATCH // _NW
_CHUNK = 8
_NCHUNK = _RPW // _CHUNK


def _rotl32(x, r):
    return ((x << np.uint32(r)) | (x >> np.uint32(32 - r))).astype(np.uint32)


def _gumbel_noise():
    n = BATCH * N_UNIT
    k1, k2 = np.uint32(0), np.uint32(42)
    ks = [k1, k2, np.uint32(k1 ^ k2 ^ np.uint32(0x1BD11BDA))]
    x = [np.zeros(n, dtype=np.uint32), np.arange(n, dtype=np.uint32)]
    rotations = [[13, 15, 26, 6], [17, 29, 16, 24]]
    x[0] += ks[0]
    x[1] += ks[1]
    for i in range(5):
        for r in rotations[i % 2]:
            x[0] += x[1]
            x[1] = _rotl32(x[1], r)
            x[1] ^= x[0]
        x[0] += ks[(i + 1) % 3]
        x[1] += ks[(i + 2) % 3] + np.uint32(i + 1)
    bits = x[0] ^ x[1]
    fl = ((bits >> np.uint32(9)) | np.uint32(0x3F800000)).view(np.float32) \
        - np.float32(1.0)
    mn, mx = np.float32(1e-6), np.float32(1.0 - 1e-6)
    u = np.maximum(mn, fl * (mx - mn) + mn).astype(np.float64)
    return (-np.log(-np.log(u))).astype(np.float32).reshape(BATCH, N_UNIT)


_G = _gumbel_noise()


def _sc_gather_body(table_hbm, idx_hbm, out_hbm, idx_v, rows_v, sems):
    wid = lax.axis_index("s") * 2 + lax.axis_index("c")
    base = pl.multiple_of(wid * _RPW, _RPW)
    pltpu.sync_copy(idx_hbm.at[pl.ds(base, _RPW)], idx_v)

    def _start(c):
        slot = c % 2
        pltpu.make_async_copy(table_hbm.at[idx_v.at[pl.ds(c * _CHUNK, _CHUNK)]],
                              rows_v.at[slot], sems.at[slot]).start()

    def _wait(c):
        slot = c % 2
        pltpu.make_async_copy(table_hbm.at[idx_v.at[pl.ds(c * _CHUNK, _CHUNK)]],
                              rows_v.at[slot], sems.at[slot]).wait()

    _start(0)
    for c in range(_NCHUNK):
        if c + 1 < _NCHUNK:
            _start(c + 1)
        _wait(c)
        pltpu.sync_copy(rows_v.at[c % 2],
                        out_hbm.at[pl.ds(base + c * _CHUNK, _CHUNK)])


def _sc_gather(B, batch_ind):
    k = pl.kernel(
        _sc_gather_body,
        out_type=jax.ShapeDtypeStruct((BATCH, N_UNIT), jnp.float32),
        mesh=plsc.VectorSubcoreMesh(core_axis_name="c", subcore_axis_name="s",
                                    num_cores=2, num_subcores=16),
        scratch_types=[
            pltpu.VMEM((_RPW,), jnp.int32),
            pltpu.VMEM((2, _CHUNK, N_UNIT), jnp.float32),
            pltpu.SemaphoreType.DMA((2,)),
        ],
    )
    return k(B, batch_ind)


def _fused_kernel(x_ref,
                  t_ref,
                  m_ref,
                  wenc_ref,
                  benc_ref,
                  bg_ref,
                  g_ref,
                  ind1_ref,
                  c0_ref,
                  yc_ref,
                  yb_ref,
                  ym_ref,
                  wdec_ref,
                  bdec_ref,
                  err_ref, reg_ref, prog_ref, rec_ref, ym_s_ref, sm_s_ref):
    i = pl.program_id(0)

    @pl.when(i == 0)
    def _init():
        for r in (err_ref, reg_ref, prog_ref, rec_ref, ym_s_ref, sm_s_ref):
            r[...] = jnp.zeros_like(r)

    xb = x_ref[...].reshape(T * _BB, D)
    tb = t_ref[...].reshape(T * _BB, 1)
    mb = m_ref[...].reshape(T * _BB, 1)
    m2 = mb * mb
    h = jnp.tanh(jnp.dot(xb, wenc_ref[...], preferred_element_type=jnp.float32)
                 + benc_ref[...] + 0.1 * tb) * mb
    sm = jnp.sum(mb.reshape(T, _BB, 1), axis=0)
    c = jnp.sum(h.reshape(T, _BB, HID), axis=0) / (sm + 1e-8)
    s1 = jnp.sum((xb * m2).reshape(T, _BB, D), axis=0)
    sq = jnp.sum(m2.reshape(T, _BB, 1), axis=0)
    rec_ref[...] += jnp.sum(xb * xb * m2, keepdims=True)
    sm_s_ref[...] += jnp.sum(sm, keepdims=True)

    x_dec = jnp.tanh(jnp.dot(c, wdec_ref[...],
                             preferred_element_type=jnp.float32) + bdec_ref[...])
    rec_ref[...] += jnp.sum(-2.0 * s1 * x_dec + sq * x_dec * x_dec,
                            keepdims=True)
    ym_s_ref[...] += jnp.sum(ym_ref[...], keepdims=True)

    logits = (bg_ref[...] + g_ref[...]) * (1.0 / TAU)
    col = jax.lax.broadcasted_iota(jnp.int32, (_BB, N_UNIT), 1)
    logits = jnp.where(col == ind1_ref[...], jnp.float32(-1e30), logits)
    mx = jnp.max(logits, axis=1, keepdims=True)
    e = jnp.exp(logits - mx)
    s = jnp.sum(e, axis=1, keepdims=True)
    p = e * (1.0 / s)
    reg_ref[...] += jnp.sum(p * p, keepdims=True)
    pc0 = jnp.dot(p, c0_ref[...], preferred_element_type=jnp.float32)
    err = c - pc0
    err_ref[...] += jnp.sum(err * err, keepdims=True)
    yh = jnp.dot(p, yc_ref[...], preferred_element_type=jnp.float32)
    dy = yb_ref[...] - yh
    prog_ref[...] += jnp.sum(dy * dy * ym_ref[...], keepdims=True)


def kernel(x, t, mask, batch_ind, y_batch, y_control, y_mask, B, C0,
           W_enc, b_enc, W_dec, b_dec):
    f32 = jnp.float32
    B_gathered = _sc_gather(B, batch_ind)

    wrap = jnp.any(batch_ind >= N_UNIT)
    ind1 = jnp.where(wrap, batch_ind % (N_UNIT + 1), batch_ind)
    ind1m = jnp.where(ind1 < N_UNIT, ind1, -1).astype(jnp.int32).reshape(BATCH, 1)

    outs = pl.pallas_call(
        _fused_kernel,
        grid=(_NSTEPS,),
        in_specs=[
            pl.BlockSpec((T, _BB, D), lambda i: (0, i, 0)),
            pl.BlockSpec((T, _BB, 1), lambda i: (0, i, 0)),
            pl.BlockSpec((T, _BB, 1), lambda i: (0, i, 0)),
            pl.BlockSpec((D, HID), lambda i: (0, 0)),
            pl.BlockSpec((1, HID), lambda i: (0, 0)),
            pl.BlockSpec((_BB, N_UNIT), lambda i: (i, 0)),
            pl.BlockSpec((_BB, N_UNIT), lambda i: (i, 0)),
            pl.BlockSpec((_BB, 1), lambda i: (i, 0)),
            pl.BlockSpec((N_UNIT, HID), lambda i: (0, 0)),
            pl.BlockSpec((N_UNIT, DY), lambda i: (0, 0)),
            pl.BlockSpec((_BB, DY), lambda i: (i, 0)),
            pl.BlockSpec((_BB, 1), lambda i: (i, 0)),
            pl.BlockSpec((HID, D), lambda i: (0, 0)),
            pl.BlockSpec((1, D), lambda i: (0, 0)),
        ],
        out_specs=[pl.BlockSpec((1, 1), lambda i: (0, 0))] * 6,
        out_shape=[jax.ShapeDtypeStruct((1, 1), f32)] * 6,
    )(x, t, mask, W_enc, b_enc.reshape(1, HID), B_gathered, _G, ind1m, C0,
      y_control, y_batch, y_mask.reshape(BATCH, 1), W_dec, b_dec.reshape(1, D))
    err_s, reg_s, prog_s, rec_s, ym_s, sm_s = [o[0, 0] for o in outs]

    err_mse = err_s / (BATCH * HID)
    reg = reg_s / (BATCH * N_UNIT)
    sel = LAM_EXPRESS * (err_mse + REG_B * reg)
    rec = rec_s / sm_s * LAM_RECON
    prog = prog_s / ym_s * LAM_PROG
    return sel + rec + prog


# speedup vs baseline: 1.1736x; 1.0383x over previous
import jax
import jax.numpy as jnp
import numpy as np
from jax import lax
from jax.experimental import pallas as pl
from jax.experimental.pallas import tpu as pltpu
from jax.experimental.pallas import tpu_sc as plsc

N_UNIT = 4096
N_TREATED = 512
BATCH = 1024
HID = 64
T = 50
D = 16
DY = 8
TAU = 1.0
REG_B = 0.1
LAM_EXPRESS = 1.0
LAM_RECON = 0.5
LAM_PROG = 1.0

_BB = 128
_NSTEPS = BATCH // _BB

_NW = 32
_RPW = BATCH // _NW
_CHUNK = 8
_NCHUNK = _RPW // _CHUNK


def _rotl32(x, r):
    return ((x << np.uint32(r)) | (x >> np.uint32(32 - r))).astype(np.uint32)


def _gumbel_noise():
    n = BATCH * N_UNIT
    k1, k2 = np.uint32(0), np.uint32(42)
    ks = [k1, k2, np.uint32(k1 ^ k2 ^ np.uint32(0x1BD11BDA))]
    x = [np.zeros(n, dtype=np.uint32), np.arange(n, dtype=np.uint32)]
    rotations = [[13, 15, 26, 6], [17, 29, 16, 24]]
    x[0] += ks[0]
    x[1] += ks[1]
    for i in range(5):
        for r in rotations[i % 2]:
            x[0] += x[1]
            x[1] = _rotl32(x[1], r)
            x[1] ^= x[0]
        x[0] += ks[(i + 1) % 3]
        x[1] += ks[(i + 2) % 3] + np.uint32(i + 1)
    bits = x[0] ^ x[1]
    fl = ((bits >> np.uint32(9)) | np.uint32(0x3F800000)).view(np.float32) \
        - np.float32(1.0)
    mn, mx = np.float32(1e-6), np.float32(1.0 - 1e-6)
    u = np.maximum(mn, fl * (mx - mn) + mn).astype(np.float64)
    return (-np.log(-np.log(u))).astype(np.float32).reshape(BATCH, N_UNIT)


_G = _gumbel_noise()


def _sc_gather_body(table_hbm, idx_hbm, out_hbm, idx_v, rows_v, sems):
    wid = lax.axis_index("s") * 2 + lax.axis_index("c")
    base = pl.multiple_of(wid * _RPW, _RPW)
    pltpu.sync_copy(idx_hbm.at[pl.ds(base, _RPW)], idx_v)

    def _start(c):
        slot = c % 2
        pltpu.make_async_copy(table_hbm.at[idx_v.at[pl.ds(c * _CHUNK, _CHUNK)]],
                              rows_v.at[slot], sems.at[slot]).start()

    def _wait(c):
        slot = c % 2
        pltpu.make_async_copy(table_hbm.at[idx_v.at[pl.ds(c * _CHUNK, _CHUNK)]],
                              rows_v.at[slot], sems.at[slot]).wait()

    _start(0)
    for c in range(_NCHUNK):
        if c + 1 < _NCHUNK:
            _start(c + 1)
        _wait(c)
        pltpu.sync_copy(rows_v.at[c % 2],
                        out_hbm.at[pl.ds(base + c * _CHUNK, _CHUNK)])


def _sc_gather(B, batch_ind):
    k = pl.kernel(
        _sc_gather_body,
        out_type=jax.ShapeDtypeStruct((BATCH, N_UNIT), jnp.float32),
        mesh=plsc.VectorSubcoreMesh(core_axis_name="c", subcore_axis_name="s",
                                    num_cores=2, num_subcores=16),
        scratch_types=[
            pltpu.VMEM((_RPW,), jnp.int32),
            pltpu.VMEM((2, _CHUNK, N_UNIT), jnp.float32),
            pltpu.SemaphoreType.DMA((2,)),
        ],
    )
    return k(B, batch_ind)


def _fused_kernel(x_ref,
                  t_ref,
                  m_ref,
                  wenc_ref,
                  benc_ref,
                  bg_ref,
                  g_ref,
                  bi_ref,
                  c0_ref,
                  yc_ref,
                  yb_ref,
                  ym_ref,
                  wdec_ref,
                  bdec_ref,
                  loss_ref,
                  err_ref, reg_ref, prog_ref, rec_ref, ym_s_ref, sm_s_ref):
    i = pl.program_id(0)

    @pl.when(i == 0)
    def _init():
        for r in (err_ref, reg_ref, prog_ref, rec_ref, ym_s_ref, sm_s_ref):
            r[...] = jnp.zeros_like(r)

    xb = x_ref[...].reshape(T * _BB, D)
    tb = t_ref[...].reshape(T * _BB, 1)
    mb = m_ref[...].reshape(T * _BB, 1)
    m2 = mb * mb
    h = jnp.tanh(jnp.dot(xb, wenc_ref[...], preferred_element_type=jnp.float32)
                 + benc_ref[...] + 0.1 * tb) * mb
    sm = jnp.sum(mb.reshape(T, _BB, 1), axis=0)
    c = jnp.sum(h.reshape(T, _BB, HID), axis=0) / (sm + 1e-8)
    s1 = jnp.sum((xb * m2).reshape(T, _BB, D), axis=0)
    sq = jnp.sum(m2.reshape(T, _BB, 1), axis=0)
    rec_ref[...] += jnp.sum(xb * xb * m2, keepdims=True)
    sm_s_ref[...] += jnp.sum(sm, keepdims=True)

    x_dec = jnp.tanh(jnp.dot(c, wdec_ref[...],
                             preferred_element_type=jnp.float32) + bdec_ref[...])
    rec_ref[...] += jnp.sum(-2.0 * s1 * x_dec + sq * x_dec * x_dec,
                            keepdims=True)
    ym_s_ref[...] += jnp.sum(ym_ref[...], keepdims=True)

    ind1 = bi_ref[...] % jnp.int32(N_UNIT + 1)
    ind1 = jnp.where(ind1 < N_UNIT, ind1, -1)
    logits = (bg_ref[...] + g_ref[...]) * (1.0 / TAU)
    col = jax.lax.broadcasted_iota(jnp.int32, (_BB, N_UNIT), 1)
    logits = jnp.where(col == ind1, jnp.float32(-1e30), logits)
    mx = jnp.max(logits, axis=1, keepdims=True)
    e = jnp.exp(logits - mx)
    s = jnp.sum(e, axis=1, keepdims=True)
    p = e * (1.0 / s)
    reg_ref[...] += jnp.sum(p * p, keepdims=True)
    pc0 = jnp.dot(p, c0_ref[...], preferred_element_type=jnp.float32)
    err = c - pc0
    err_ref[...] += jnp.sum(err * err, keepdims=True)
    yh = jnp.dot(p, yc_ref[...], preferred_element_type=jnp.float32)
    dy = yb_ref[...] - yh
    prog_ref[...] += jnp.sum(dy * dy * ym_ref[...], keepdims=True)

    @pl.when(i == _NSTEPS - 1)
    def _fin():
        err_mse = err_ref[...] * (1.0 / (BATCH * HID))
        reg = reg_ref[...] * (1.0 / (BATCH * N_UNIT))
        sel = LAM_EXPRESS * (err_mse + REG_B * reg)
        rec = rec_ref[...] / sm_s_ref[...] * LAM_RECON
        prog = prog_ref[...] / ym_s_ref[...] * LAM_PROG
        loss_ref[...] = sel + rec + prog


def kernel(x, t, mask, batch_ind, y_batch, y_control, y_mask, B, C0,
           W_enc, b_enc, W_dec, b_dec):
    f32 = jnp.float32
    B_gathered = _sc_gather(B, batch_ind)

    outs = pl.pallas_call(
        _fused_kernel,
        grid=(_NSTEPS,),
        in_specs=[
            pl.BlockSpec((T, _BB, D), lambda i: (0, i, 0)),
            pl.BlockSpec((T, _BB, 1), lambda i: (0, i, 0)),
            pl.BlockSpec((T, _BB, 1), lambda i: (0, i, 0)),
            pl.BlockSpec((D, HID), lambda i: (0, 0)),
            pl.BlockSpec((1, HID), lambda i: (0, 0)),
            pl.BlockSpec((_BB, N_UNIT), lambda i: (i, 0)),
            pl.BlockSpec((_BB, N_UNIT), lambda i: (i, 0)),
            pl.BlockSpec((_BB, 1), lambda i: (i, 0)),
            pl.BlockSpec((N_UNIT, HID), lambda i: (0, 0)),
            pl.BlockSpec((N_UNIT, DY), lambda i: (0, 0)),
            pl.BlockSpec((_BB, DY), lambda i: (i, 0)),
            pl.BlockSpec((_BB, 1), lambda i: (i, 0)),
            pl.BlockSpec((HID, D), lambda i: (0, 0)),
            pl.BlockSpec((1, D), lambda i: (0, 0)),
        ],
        out_specs=[pl.BlockSpec((1, 1), lambda i: (0, 0))] * 7,
        out_shape=[jax.ShapeDtypeStruct((1, 1), f32)] * 7,
    )(x, t, mask, W_enc, b_enc.reshape(1, HID), B_gathered, _G,
      batch_ind.reshape(BATCH, 1), C0, y_control, y_batch,
      y_mask.reshape(BATCH, 1), W_dec, b_dec.reshape(1, D))
    return outs[0][0, 0]


# speedup vs baseline: 1.1998x; 1.0223x over previous
import jax
import jax.numpy as jnp
import numpy as np
from jax import lax
from jax.experimental import pallas as pl
from jax.experimental.pallas import tpu as pltpu
from jax.experimental.pallas import tpu_sc as plsc

N_UNIT = 4096
N_TREATED = 512
BATCH = 1024
HID = 64
T = 50
D = 16
DY = 8
TAU = 1.0
REG_B = 0.1
LAM_EXPRESS = 1.0
LAM_RECON = 0.5
LAM_PROG = 1.0

_BB = 128
_NSTEPS = BATCH // _BB

_NW = 32
_RPW = BATCH // _NW
_CHUNK = 8
_NCHUNK = _RPW // _CHUNK


def _rotl32(x, r):
    return ((x << np.uint32(r)) | (x >> np.uint32(32 - r))).astype(np.uint32)


def _gumbel_noise():
    n = BATCH * N_UNIT
    k1, k2 = np.uint32(0), np.uint32(42)
    ks = [k1, k2, np.uint32(k1 ^ k2 ^ np.uint32(0x1BD11BDA))]
    x = [np.zeros(n, dtype=np.uint32), np.arange(n, dtype=np.uint32)]
    rotations = [[13, 15, 26, 6], [17, 29, 16, 24]]
    x[0] += ks[0]
    x[1] += ks[1]
    for i in range(5):
        for r in rotations[i % 2]:
            x[0] += x[1]
            x[1] = _rotl32(x[1], r)
            x[1] ^= x[0]
        x[0] += ks[(i + 1) % 3]
        x[1] += ks[(i + 2) % 3] + np.uint32(i + 1)
    bits = x[0] ^ x[1]
    fl = ((bits >> np.uint32(9)) | np.uint32(0x3F800000)).view(np.float32) \
        - np.float32(1.0)
    mn, mx = np.float32(1e-6), np.float32(1.0 - 1e-6)
    u = np.maximum(mn, fl * (mx - mn) + mn).astype(np.float64)
    return (-np.log(-np.log(u))).astype(np.float32).reshape(BATCH, N_UNIT)


_G = _gumbel_noise()


def _sc_gather_body(table_hbm, idx_hbm, out_hbm, idx_v, rows_v, sems):
    wid = lax.axis_index("s") * 2 + lax.axis_index("c")
    base = pl.multiple_of(wid * _RPW, _RPW)
    pltpu.sync_copy(idx_hbm.at[pl.ds(base, _RPW)], idx_v)

    def _start(c):
        slot = c % 2
        pltpu.make_async_copy(table_hbm.at[idx_v.at[pl.ds(c * _CHUNK, _CHUNK)]],
                              rows_v.at[slot], sems.at[slot]).start()

    def _wait(c):
        slot = c % 2
        pltpu.make_async_copy(table_hbm.at[idx_v.at[pl.ds(c * _CHUNK, _CHUNK)]],
                              rows_v.at[slot], sems.at[slot]).wait()

    _start(0)
    for c in range(_NCHUNK):
        if c + 1 < _NCHUNK:
            _start(c + 1)
        _wait(c)
        pltpu.sync_copy(rows_v.at[c % 2],
                        out_hbm.at[pl.ds(base + c * _CHUNK, _CHUNK)])


def _sc_gather(B, batch_ind):
    k = pl.kernel(
        _sc_gather_body,
        out_type=jax.ShapeDtypeStruct((BATCH, N_UNIT), jnp.float32),
        mesh=plsc.VectorSubcoreMesh(core_axis_name="c", subcore_axis_name="s",
                                    num_cores=2, num_subcores=16),
        scratch_types=[
            pltpu.VMEM((_RPW,), jnp.int32),
            pltpu.VMEM((2, _CHUNK, N_UNIT), jnp.float32),
            pltpu.SemaphoreType.DMA((2,)),
        ],
    )
    return k(B, batch_ind)


def _fused_kernel(x_ref,
                  t_ref,
                  m_ref,
                  wenc_ref,
                  benc_ref,
                  bg_ref,
                  g_ref,
                  bi_ref,
                  c0_ref,
                  yc_ref,
                  yb_ref,
                  ym_ref,
                  wdec_ref,
                  bdec_ref,
                  loss_ref,
                  err_ref, reg_ref, prog_ref, rec_ref, ym_s_ref, sm_s_ref):
    i = pl.program_id(0)

    @pl.when(i == 0)
    def _init():
        for r in (err_ref, reg_ref, prog_ref, rec_ref, ym_s_ref, sm_s_ref):
            r[...] = jnp.zeros_like(r)

    xr = x_ref[...]
    tr = t_ref[...]
    mr = m_ref[...]
    wenc = wenc_ref[...]
    benc = benc_ref[...]
    csum = jnp.zeros((_BB, HID), jnp.float32)
    s1 = jnp.zeros((_BB, D), jnp.float32)
    s2v = jnp.zeros((_BB, D), jnp.float32)
    for tt in range(T):
        xt = xr[:, tt * D:(tt + 1) * D]
        mt = mr[:, tt:tt + 1]
        tv = tr[:, tt:tt + 1]
        z = jnp.dot(xt, wenc, preferred_element_type=jnp.float32) \
            + benc + 0.1 * tv
        csum += jnp.tanh(z) * mt
        m2t = mt * mt
        s1 += xt * m2t
        s2v += xt * xt * m2t
    sm = jnp.sum(mr, axis=1, keepdims=True)
    sq = jnp.sum(mr * mr, axis=1, keepdims=True)
    c = csum / (sm + 1e-8)
    rec_ref[...] += jnp.sum(s2v, keepdims=True)
    sm_s_ref[...] += jnp.sum(sm, keepdims=True)

    x_dec = jnp.tanh(jnp.dot(c, wdec_ref[...],
                             preferred_element_type=jnp.float32) + bdec_ref[...])
    rec_ref[...] += jnp.sum(-2.0 * s1 * x_dec + sq * x_dec * x_dec,
                            keepdims=True)
    ym_s_ref[...] += jnp.sum(ym_ref[...], keepdims=True)

    ind1 = bi_ref[...] % jnp.int32(N_UNIT + 1)
    ind1 = jnp.where(ind1 < N_UNIT, ind1, -1)
    logits = (bg_ref[...] + g_ref[...]) * (1.0 / TAU)
    col = jax.lax.broadcasted_iota(jnp.int32, (_BB, N_UNIT), 1)
    logits = jnp.where(col == ind1, jnp.float32(-1e30), logits)
    mx = jnp.max(logits, axis=1, keepdims=True)
    e = jnp.exp(logits - mx)
    s = jnp.sum(e, axis=1, keepdims=True)
    p = e * (1.0 / s)
    reg_ref[...] += jnp.sum(p * p, keepdims=True)
    pc0 = jnp.dot(p, c0_ref[...], preferred_element_type=jnp.float32)
    err = c - pc0
    err_ref[...] += jnp.sum(err * err, keepdims=True)
    yh = jnp.dot(p, yc_ref[...], preferred_element_type=jnp.float32)
    dy = yb_ref[...] - yh
    prog_ref[...] += jnp.sum(dy * dy * ym_ref[...], keepdims=True)

    @pl.when(i == _NSTEPS - 1)
    def _fin():
        err_mse = err_ref[...] * (1.0 / (BATCH * HID))
        reg = reg_ref[...] * (1.0 / (BATCH * N_UNIT))
        sel = LAM_EXPRESS * (err_mse + REG_B * reg)
        rec = rec_ref[...] / sm_s_ref[...] * LAM_RECON
        prog = prog_ref[...] / ym_s_ref[...] * LAM_PROG
        loss_ref[...] = sel + rec + prog


def kernel(x, t, mask, batch_ind, y_batch, y_control, y_mask, B, C0,
           W_enc, b_enc, W_dec, b_dec):
    f32 = jnp.float32
    B_gathered = _sc_gather(B, batch_ind)

    xt_bm = x.transpose(1, 0, 2).reshape(BATCH, T * D)
    t_bm = t.reshape(T, BATCH).T
    m_bm = mask.reshape(T, BATCH).T

    outs = pl.pallas_call(
        _fused_kernel,
        grid=(_NSTEPS,),
        in_specs=[
            pl.BlockSpec((_BB, T * D), lambda i: (i, 0)),
            pl.BlockSpec((_BB, T), lambda i: (i, 0)),
            pl.BlockSpec((_BB, T), lambda i: (i, 0)),
            pl.BlockSpec((D, HID), lambda i: (0, 0)),
            pl.BlockSpec((1, HID), lambda i: (0, 0)),
            pl.BlockSpec((_BB, N_UNIT), lambda i: (i, 0)),
            pl.BlockSpec((_BB, N_UNIT), lambda i: (i, 0)),
            pl.BlockSpec((_BB, 1), lambda i: (i, 0)),
            pl.BlockSpec((N_UNIT, HID), lambda i: (0, 0)),
            pl.BlockSpec((N_UNIT, DY), lambda i: (0, 0)),
            pl.BlockSpec((_BB, DY), lambda i: (i, 0)),
            pl.BlockSpec((_BB, 1), lambda i: (i, 0)),
            pl.BlockSpec((HID, D), lambda i: (0, 0)),
            pl.BlockSpec((1, D), lambda i: (0, 0)),
        ],
        out_specs=[pl.BlockSpec((1, 1), lambda i: (0, 0))] * 7,
        out_shape=[jax.ShapeDtypeStruct((1, 1), f32)] * 7,
    )(xt_bm, t_bm, m_bm, W_enc, b_enc.reshape(1, HID), B_gathered, _G,
      batch_ind.reshape(BATCH, 1), C0, y_control, y_batch,
      y_mask.reshape(BATCH, 1), W_dec, b_dec.reshape(1, D))
    return outs[0][0, 0]
